# baseline (speedup 1.0000x reference)
import functools

import jax
import jax.numpy as jnp
import numpy as np
from jax import lax
from jax.experimental import pallas as pl
from jax.experimental.pallas import tpu as pltpu
from jax.experimental.pallas import tpu_sc as plsc

_N = 262144
_NC, _NS, _L = 2, 16, 16
_NW = _NC * _NS
_RPW = _N // _NW
_C = 64
_NCH = _RPW // _C
_S = 48
_RADIUS = 500.0

_LEVELS = (
    (16, 0, 4096, False),
    (32, 4096, 32768, False),
    (64, 36864, 262144, False),
    (128, 299008, 524288, True),
    (256, 823296, 524288, True),
    (512, 1347584, 524288, True),
)
_P1 = np.int32(2654435761 - (1 << 32))
_P2 = np.int32(805459861)


def _coords_body(vd_ref, ro_ref, out_ref):
    vd = vd_ref[...]
    ro = ro_ref[...]
    b = 2.0 * jnp.sum(vd * ro, axis=0, keepdims=True)
    c = jnp.sum(ro * ro, axis=0, keepdims=True) - _RADIUS * _RADIUS
    disc = b * b - 4.0 * c
    sq = jnp.sqrt(jnp.maximum(disc, 0.0))
    t = 0.5 * (sq - b)
    pts = ro + t * vd
    out_ref[...] = jnp.clip((pts + _RADIUS) / (2.0 * _RADIUS), 0.0, 1.0)


_BT = 8192


def _coords_tc(vd_t, ro_t):
    return pl.pallas_call(
        _coords_body,
        grid=(_N // _BT,),
        in_specs=[
            pl.BlockSpec((3, _BT), lambda i: (0, i)),
            pl.BlockSpec((3, _BT), lambda i: (0, i)),
        ],
        out_specs=pl.BlockSpec((3, _BT), lambda i: (0, i)),
        out_shape=jax.ShapeDtypeStruct((3, _N), jnp.float32),
    )(vd_t, ro_t)


_NT = 14624
_TPW = _NT // _NW
_TCH = 16
_NFULL = _TPW // _TCH
_TTAIL = _TPW - _NFULL * _TCH


def _ilv_body(tp_hbm, out_hbm, in0, in1, ov0, ov1, isem0, isem1, osem0, osem1):
    wid = lax.axis_index("s") * _NC + lax.axis_index("c")
    k0 = wid * _TPW
    lane = lax.iota(jnp.int32, _L)
    in_b = (in0, in1)
    ov_b = (ov0, ov1)
    isem_b = (isem0, isem1)
    osem_b = (osem0, osem1)

    def src(ch, nt):
        return tp_hbm.at[pl.ds((k0 + ch * _TCH) * 512, nt * 512)]

    def dst(ch, nt, buf):
        return out_hbm.at[pl.ds((k0 + ch * _TCH) * 512, nt * 512)]

    def ilv(buf, nt):
        in_v = in_b[buf]
        ov = ov_b[buf]

        def tile_body(t, c2):
            t512 = t * 512
            for c8 in range(8):
                ibase = t512 + (c8 & 3) * 128 + (c8 >> 2) * 64
                obase = t512 + 8 * lane + c8
                for j in range(4):
                    v = in_v[pl.ds(ibase + 16 * j, _L)]
                    plsc.store_scatter(ov, [obase + 128 * j], v)
            return c2

        lax.fori_loop(0, nt, tile_body, 0)

    def prep(ch, buf, nt):
        pltpu.async_copy(src(ch, nt), in_b[buf].at[pl.ds(0, nt * 512)],
                         isem_b[buf])

    def wait_in(ch, buf, nt):
        pltpu.make_async_copy(src(ch, nt), in_b[buf].at[pl.ds(0, nt * 512)],
                              isem_b[buf]).wait()

    def put(ch, buf, nt):
        pltpu.async_copy(ov_b[buf].at[pl.ds(0, nt * 512)], dst(ch, nt, buf),
                         osem_b[buf])

    def wait_put(ch, buf, nt):
        pltpu.make_async_copy(ov_b[buf].at[pl.ds(0, nt * 512)],
                              dst(ch, nt, buf), osem_b[buf]).wait()

    prep(0, 0, _TCH)

    def pair_body(h, carry):
        g0 = 2 * h
        prep(g0 + 1, 1, _TCH)
        wait_in(g0, 0, _TCH)

        @pl.when(h >= 1)
        def _():
            wait_put(g0 - 2, 0, _TCH)

        ilv(0, _TCH)
        put(g0, 0, _TCH)

        @pl.when(h < _NFULL // 2 - 1)
        def _():
            prep(g0 + 2, 0, _TCH)

        wait_in(g0 + 1, 1, _TCH)

        @pl.when(h >= 1)
        def _():
            wait_put(g0 - 1, 1, _TCH)

        ilv(1, _TCH)
        put(g0 + 1, 1, _TCH)
        return carry

    lax.fori_loop(0, _NFULL // 2, pair_body, 0)
    wait_put(_NFULL - 2, 0, _TCH)
    wait_put(_NFULL - 1, 1, _TCH)
    prep(_NFULL, 0, _TTAIL)
    wait_in(_NFULL, 0, _TTAIL)
    ilv(0, _TTAIL)
    put(_NFULL, 0, _TTAIL)
    wait_put(_NFULL, 0, _TTAIL)


_ilv_sc = functools.partial(
    pl.kernel,
    out_type=jax.ShapeDtypeStruct((_NT * 512,), jnp.float32),
    mesh=plsc.VectorSubcoreMesh(
        core_axis_name="c", subcore_axis_name="s", num_cores=_NC, num_subcores=_NS
    ),
    compiler_params=pltpu.CompilerParams(
        needs_layout_passes=False, use_tc_tiling_on_sc=False
    ),
    scratch_types=[
        pltpu.VMEM((_TCH * 512,), jnp.float32),
        pltpu.VMEM((_TCH * 512,), jnp.float32),
        pltpu.VMEM((_TCH * 512,), jnp.float32),
        pltpu.VMEM((_TCH * 512,), jnp.float32),
        pltpu.SemaphoreType.DMA,
        pltpu.SemaphoreType.DMA,
        pltpu.SemaphoreType.DMA,
        pltpu.SemaphoreType.DMA,
    ],
)(_ilv_body)


def _sc_body(xs_hbm, ys_hbm, zs_hbm, table_hbm, out_hbm,
             xs_v, ys_v, zs_v, idx0, idx1, w0, w1, rows0, rows1,
             out0, out1, gsem0, gsem1, osem0, osem1):
    wid = lax.axis_index("s") * _NC + lax.axis_index("c")
    lane = lax.iota(jnp.int32, _L)
    rep4 = lane >> 2
    col = lane & 3

    base = wid * _RPW
    pltpu.sync_copy(xs_hbm.at[pl.ds(base, _RPW)], xs_v)
    pltpu.sync_copy(ys_hbm.at[pl.ds(base, _RPW)], ys_v)
    pltpu.sync_copy(zs_hbm.at[pl.ds(base, _RPW)], zs_v)

    idx_b = (idx0, idx1)
    w_b = (w0, w1)
    rows_b = (rows0, rows1)
    out_b = (out0, out1)
    gsem_b = (gsem0, gsem1)
    osem_b = (osem0, osem1)

    def prep(ch, buf):
        idx_v = idx_b[buf]
        w_v = w_b[buf]

        def iw_body(i, c2):
            o = i * _L
            og = ch * _C + o
            x = xs_v[pl.ds(og, _L)]
            y = ys_v[pl.ds(og, _L)]
            z = zs_v[pl.ds(og, _L)]
            for l, (R, off, size, hashed) in enumerate(_LEVELS):
                scale = jnp.float32(R - 1)
                px = x * scale
                py = y * scale
                pz = z * scale
                cx = jnp.minimum(px.astype(jnp.int32), R - 2)
                cy = jnp.minimum(py.astype(jnp.int32), R - 2)
                cz = jnp.minimum(pz.astype(jnp.int32), R - 2)
                fx = px - cx.astype(jnp.float32)
                fy = py - cy.astype(jnp.float32)
                fz = pz - cz.astype(jnp.float32)
                wx = (1.0 - fx, fx)
                wy = (1.0 - fy, fy)
                wz = (1.0 - fz, fz)
                if hashed:
                    hy0 = cy * _P1
                    hz0 = cz * _P2
                    hx = (cx, cx + 1)
                    hy = (hy0, hy0 + _P1)
                    hz = (hz0, hz0 + _P2)
                    mask = size - 1
                    corner_idx = [
                        ((hx[c & 1] ^ hy[(c >> 1) & 1] ^ hz[(c >> 2) & 1]) & mask)
                        + off
                        for c in range(8)
                    ]
                else:
                    by0 = cy * R
                    bz0 = cz * (R * R)
                    bx = (cx, cx + 1)
                    by = (by0, by0 + R)
                    bz = (bz0, bz0 + R * R)
                    corner_idx = [
                        bx[c & 1] + by[(c >> 1) & 1] + bz[(c >> 2) & 1] + off
                        for c in range(8)
                    ]
                for corner in range(8):
                    dx, dy, dz = corner & 1, (corner >> 1) & 1, (corner >> 2) & 1
                    idx = corner_idx[corner]
                    wgt = wx[dx] * wy[dy] * wz[dz]
                    wbits = lax.bitcast_convert_type(wgt, jnp.int32)
                    wbits = wbits | lax.shift_left(idx & 64, 25)
                    slot = l * 8 + corner
                    idx_v[pl.ds(slot * _C + o, _L)] = (
                        lax.shift_right_logical(idx, 1) & -64
                    ) | (idx & 63)
                    w_v[pl.ds(slot * _C + o, _L)] = lax.bitcast_convert_type(
                        wbits, jnp.float32
                    )
            return c2

        lax.fori_loop(0, _C // _L, iw_body, 0)

        pltpu.async_copy(table_hbm.at[idx_v], rows_b[buf], gsem_b[buf])

    def wait_gather(buf):
        pltpu.make_async_copy(
            table_hbm.at[idx_b[buf]], rows_b[buf], gsem_b[buf]
        ).wait()

    def acc(ch, buf):
        rows_v = rows_b[buf]
        w_v = w_b[buf]
        out_v = out_b[buf]

        def acc_body(j, c2):
            r4 = 4 * j + rep4
            for l in range(6):
                a = jnp.zeros((_L,), jnp.float32)
                for corner in range(8):
                    s = l * 8 + corner
                    wraw = plsc.load_gather(w_v, [s * _C + r4])
                    wbits = lax.bitcast_convert_type(wraw, jnp.int32)
                    par4 = lax.shift_left(lax.shift_right_logical(wbits, 31), 2)
                    rv = plsc.load_gather(rows_v, [s * _C + r4, col + par4])
                    a = a + jnp.abs(wraw) * rv
                plsc.store_scatter(out_v, [4 * l + col, r4], a)
            return c2

        lax.fori_loop(0, _C // 4, acc_body, 0)
        pltpu.async_copy(
            out_v, out_hbm.at[:, pl.ds(base + ch * _C, _C)], osem_b[buf]
        )

    def wait_out(ch, buf):
        pltpu.make_async_copy(
            out_b[buf], out_hbm.at[:, pl.ds(base + ch * _C, _C)], osem_b[buf]
        ).wait()

    prep(0, 0)

    def pair_body(h, carry):
        g0 = 2 * h
        prep(g0 + 1, 1)
        wait_gather(0)

        @pl.when(h >= 1)
        def _():
            wait_out(g0 - 2, 0)

        acc(g0, 0)

        @pl.when(h < _NCH // 2 - 1)
        def _():
            prep(g0 + 2, 0)

        wait_gather(1)

        @pl.when(h >= 1)
        def _():
            wait_out(g0 - 1, 1)

        acc(g0 + 1, 1)
        return carry

    lax.fori_loop(0, _NCH // 2, pair_body, 0)
    wait_out(_NCH - 2, 0)
    wait_out(_NCH - 1, 1)


_sc_hash = functools.partial(
    pl.kernel,
    out_type=jax.ShapeDtypeStruct((24, _N), jnp.float32),
    mesh=plsc.VectorSubcoreMesh(
        core_axis_name="c", subcore_axis_name="s", num_cores=_NC, num_subcores=_NS
    ),
    compiler_params=pltpu.CompilerParams(
        needs_layout_passes=False, use_tc_tiling_on_sc=False
    ),
    scratch_types=[
        pltpu.VMEM((_RPW,), jnp.float32),
        pltpu.VMEM((_RPW,), jnp.float32),
        pltpu.VMEM((_RPW,), jnp.float32),
        pltpu.VMEM((_S * _C,), jnp.int32),
        pltpu.VMEM((_S * _C,), jnp.int32),
        pltpu.VMEM((_S * _C,), jnp.float32),
        pltpu.VMEM((_S * _C,), jnp.float32),
        pltpu.VMEM((_S * _C, 8), jnp.float32),
        pltpu.VMEM((_S * _C, 8), jnp.float32),
        pltpu.VMEM((24, _C), jnp.float32),
        pltpu.VMEM((24, _C), jnp.float32),
        pltpu.SemaphoreType.DMA,
        pltpu.SemaphoreType.DMA,
        pltpu.SemaphoreType.DMA,
        pltpu.SemaphoreType.DMA,
    ],
)(_sc_body)


def kernel(view_dirs, ray_origins, table):
    coords = _coords_tc(view_dirs.T, ray_origins.T)
    xs, ys, zs = coords[0], coords[1], coords[2]
    tp = table.T.reshape(4, _NT, 128).transpose(1, 0, 2).reshape(_NT * 512)
    table2 = _ilv_sc(tp).reshape(_NT * 64, 8)
    out = _sc_hash(xs, ys, zs, table2)
    return out.T


# speedup vs baseline: 1.2992x; 1.2992x over previous
import functools

import jax
import jax.numpy as jnp
import numpy as np
from jax import lax
from jax.experimental import pallas as pl
from jax.experimental.pallas import tpu as pltpu
from jax.experimental.pallas import tpu_sc as plsc

_N = 262144
_NC, _NS, _L = 2, 16, 16
_NW = _NC * _NS
_RPW = _N // _NW
_C = 64
_NCH = _RPW // _C
_S = 48
_RADIUS = 500.0

_LEVELS = (
    (16, 0, 4096, False),
    (32, 4096, 32768, False),
    (64, 36864, 262144, False),
    (128, 299008, 524288, True),
    (256, 823296, 524288, True),
    (512, 1347584, 524288, True),
)
_P1 = np.int32(2654435761 - (1 << 32))
_P2 = np.int32(805459861)

_GSLOT = {}
_PSLOT = {}
_NG = 0
for _l, (_R, _off, _size, _hashed) in enumerate(_LEVELS):
    if _hashed:
        for _corner in range(8):
            _GSLOT[(_l, _corner)] = (_NG + _corner, 0)
        _NG += 8
    else:
        for _p in range(4):
            _PSLOT[(_l, _p)] = _NG + _p
            _GSLOT[(_l, 2 * _p)] = (_NG + _p, 0)
            _GSLOT[(_l, 2 * _p + 1)] = (_NG + _p, 4)
        _NG += 4


def _coords_body(vd_ref, ro_ref, out_ref):
    vd = vd_ref[...]
    ro = ro_ref[...]
    b = 2.0 * jnp.sum(vd * ro, axis=0, keepdims=True)
    c = jnp.sum(ro * ro, axis=0, keepdims=True) - _RADIUS * _RADIUS
    disc = b * b - 4.0 * c
    sq = jnp.sqrt(jnp.maximum(disc, 0.0))
    t = 0.5 * (sq - b)
    pts = ro + t * vd
    out_ref[...] = jnp.clip((pts + _RADIUS) / (2.0 * _RADIUS), 0.0, 1.0)


_BT = 8192


def _coords_tc(vd_t, ro_t):
    return pl.pallas_call(
        _coords_body,
        grid=(_N // _BT,),
        in_specs=[
            pl.BlockSpec((3, _BT), lambda i: (0, i)),
            pl.BlockSpec((3, _BT), lambda i: (0, i)),
        ],
        out_specs=pl.BlockSpec((3, _BT), lambda i: (0, i)),
        out_shape=jax.ShapeDtypeStruct((3, _N), jnp.float32),
    )(vd_t, ro_t)


_NT = 14624
_TPW = _NT // _NW
_TCH = 16
_NFULL = _TPW // _TCH
_TTAIL = _TPW - _NFULL * _TCH


def _ilv_body(tp_hbm, out_hbm, in0, in1, ov0, ov1, isem0, isem1, osem0, osem1):
    wid = lax.axis_index("s") * _NC + lax.axis_index("c")
    k0 = wid * _TPW
    lane = lax.iota(jnp.int32, _L)
    in_b = (in0, in1)
    ov_b = (ov0, ov1)
    isem_b = (isem0, isem1)
    osem_b = (osem0, osem1)

    def src(ch, nt):
        return tp_hbm.at[pl.ds((k0 + ch * _TCH) * 512, nt * 512)]

    def srcx(ch, nt):
        knext = jnp.minimum(k0 + ch * _TCH + nt, _NT - 1)
        return tp_hbm.at[pl.ds(knext * 512, 512)]

    def dst(ch, nt, buf):
        return out_hbm.at[pl.ds((k0 + ch * _TCH) * 1024, nt * 1024)]

    def ilv(buf, nt):
        in_v = in_b[buf]
        ov = ov_b[buf]

        def tile_body(t, c2):
            t512 = t * 512
            o512 = t * 1024
            for c in range(4):
                ibase = t512 + c * 128
                obase = o512 + 8 * lane + c
                nxt = in_v[pl.ds(t512 + 512 + c * 128, _L)]
                nxt0 = jnp.full((_L,), nxt[0], jnp.float32)
                for j in range(8):
                    v = in_v[pl.ds(ibase + 16 * j, _L)]
                    plsc.store_scatter(ov, [obase + 128 * j], v)
                    v2 = in_v[pl.ds(ibase + 16 * j + 1, _L)]
                    if j == 7:
                        v2 = jnp.where(lane == 15, nxt0, v2)
                    plsc.store_scatter(ov, [obase + 128 * j + 4], v2)
            return c2

        lax.fori_loop(0, nt, tile_body, 0)

    def prep(ch, buf, nt):
        pltpu.async_copy(src(ch, nt), in_b[buf].at[pl.ds(0, nt * 512)],
                         isem_b[buf])
        pltpu.async_copy(srcx(ch, nt), in_b[buf].at[pl.ds(nt * 512, 512)],
                         isem_b[buf])

    def wait_in(ch, buf, nt):
        pltpu.make_async_copy(src(ch, nt), in_b[buf].at[pl.ds(0, nt * 512)],
                              isem_b[buf]).wait()
        pltpu.make_async_copy(srcx(ch, nt), in_b[buf].at[pl.ds(nt * 512, 512)],
                              isem_b[buf]).wait()

    def put(ch, buf, nt):
        pltpu.async_copy(ov_b[buf].at[pl.ds(0, nt * 1024)], dst(ch, nt, buf),
                         osem_b[buf])

    def wait_put(ch, buf, nt):
        pltpu.make_async_copy(ov_b[buf].at[pl.ds(0, nt * 1024)],
                              dst(ch, nt, buf), osem_b[buf]).wait()

    prep(0, 0, _TCH)

    def pair_body(h, carry):
        g0 = 2 * h
        prep(g0 + 1, 1, _TCH)
        wait_in(g0, 0, _TCH)

        @pl.when(h >= 1)
        def _():
            wait_put(g0 - 2, 0, _TCH)

        ilv(0, _TCH)
        put(g0, 0, _TCH)

        @pl.when(h < _NFULL // 2 - 1)
        def _():
            prep(g0 + 2, 0, _TCH)

        wait_in(g0 + 1, 1, _TCH)

        @pl.when(h >= 1)
        def _():
            wait_put(g0 - 1, 1, _TCH)

        ilv(1, _TCH)
        put(g0 + 1, 1, _TCH)
        return carry

    lax.fori_loop(0, _NFULL // 2, pair_body, 0)
    wait_put(_NFULL - 2, 0, _TCH)
    wait_put(_NFULL - 1, 1, _TCH)
    prep(_NFULL, 0, _TTAIL)
    wait_in(_NFULL, 0, _TTAIL)
    ilv(0, _TTAIL)
    put(_NFULL, 0, _TTAIL)
    wait_put(_NFULL, 0, _TTAIL)


_ilv_sc = functools.partial(
    pl.kernel,
    out_type=jax.ShapeDtypeStruct((_NT * 1024,), jnp.float32),
    mesh=plsc.VectorSubcoreMesh(
        core_axis_name="c", subcore_axis_name="s", num_cores=_NC, num_subcores=_NS
    ),
    compiler_params=pltpu.CompilerParams(
        needs_layout_passes=False, use_tc_tiling_on_sc=False
    ),
    scratch_types=[
        pltpu.VMEM(((_TCH + 1) * 512,), jnp.float32),
        pltpu.VMEM(((_TCH + 1) * 512,), jnp.float32),
        pltpu.VMEM((_TCH * 1024,), jnp.float32),
        pltpu.VMEM((_TCH * 1024,), jnp.float32),
        pltpu.SemaphoreType.DMA,
        pltpu.SemaphoreType.DMA,
        pltpu.SemaphoreType.DMA,
        pltpu.SemaphoreType.DMA,
    ],
)(_ilv_body)


def _sc_body(xs_hbm, ys_hbm, zs_hbm, table_hbm, out_hbm,
             xs_v, ys_v, zs_v, idx0, idx1, w0, w1, rows0, rows1,
             out0, out1, gsem0, gsem1, osem0, osem1):
    wid = lax.axis_index("s") * _NC + lax.axis_index("c")
    lane = lax.iota(jnp.int32, _L)
    rep4 = lane >> 2
    col = lane & 3

    base = wid * _RPW
    pltpu.sync_copy(xs_hbm.at[pl.ds(base, _RPW)], xs_v)
    pltpu.sync_copy(ys_hbm.at[pl.ds(base, _RPW)], ys_v)
    pltpu.sync_copy(zs_hbm.at[pl.ds(base, _RPW)], zs_v)

    idx_b = (idx0, idx1)
    w_b = (w0, w1)
    rows_b = (rows0, rows1)
    out_b = (out0, out1)
    gsem_b = (gsem0, gsem1)
    osem_b = (osem0, osem1)

    def prep(ch, buf):
        idx_v = idx_b[buf]
        w_v = w_b[buf]

        def iw_body(i, c2):
            o = i * _L
            og = ch * _C + o
            x = xs_v[pl.ds(og, _L)]
            y = ys_v[pl.ds(og, _L)]
            z = zs_v[pl.ds(og, _L)]
            for l, (R, off, size, hashed) in enumerate(_LEVELS):
                scale = jnp.float32(R - 1)
                px = x * scale
                py = y * scale
                pz = z * scale
                cx = jnp.minimum(px.astype(jnp.int32), R - 2)
                cy = jnp.minimum(py.astype(jnp.int32), R - 2)
                cz = jnp.minimum(pz.astype(jnp.int32), R - 2)
                fx = px - cx.astype(jnp.float32)
                fy = py - cy.astype(jnp.float32)
                fz = pz - cz.astype(jnp.float32)
                wx = (1.0 - fx, fx)
                wy = (1.0 - fy, fy)
                wz = (1.0 - fz, fz)
                if hashed:
                    hy0 = cy * _P1
                    hz0 = cz * _P2
                    hx = (cx, cx + 1)
                    hy = (hy0, hy0 + _P1)
                    hz = (hz0, hz0 + _P2)
                    mask = size - 1
                    for corner in range(8):
                        idx = (
                            (hx[corner & 1]
                             ^ hy[(corner >> 1) & 1]
                             ^ hz[(corner >> 2) & 1]) & mask
                        ) + off
                        gslot = _GSLOT[(l, corner)][0]
                        idx_v[pl.ds(gslot * _C + o, _L)] = idx
                else:
                    by0 = cy * R
                    bz0 = cz * (R * R)
                    by = (by0, by0 + R)
                    bz = (bz0, bz0 + R * R)
                    for p in range(4):
                        jp = cx + by[p & 1] + bz[(p >> 1) & 1] + off
                        idx_v[pl.ds(_PSLOT[(l, p)] * _C + o, _L)] = jp
                for corner in range(8):
                    dx, dy, dz = corner & 1, (corner >> 1) & 1, (corner >> 2) & 1
                    slot = l * 8 + corner
                    w_v[pl.ds(slot * _C + o, _L)] = wx[dx] * wy[dy] * wz[dz]
            return c2

        lax.fori_loop(0, _C // _L, iw_body, 0)

        pltpu.async_copy(table_hbm.at[idx_v], rows_b[buf], gsem_b[buf])

    def wait_gather(buf):
        pltpu.make_async_copy(
            table_hbm.at[idx_b[buf]], rows_b[buf], gsem_b[buf]
        ).wait()

    def acc(ch, buf):
        rows_v = rows_b[buf]
        w_v = w_b[buf]
        out_v = out_b[buf]

        def acc_body(j, c2):
            r4 = 4 * j + rep4
            for l in range(6):
                a = jnp.zeros((_L,), jnp.float32)
                for corner in range(8):
                    s = l * 8 + corner
                    gslot, coff = _GSLOT[(l, corner)]
                    wv = plsc.load_gather(w_v, [s * _C + r4])
                    rv = plsc.load_gather(rows_v, [gslot * _C + r4, col + coff])
                    a = a + wv * rv
                plsc.store_scatter(out_v, [4 * l + col, r4], a)
            return c2

        lax.fori_loop(0, _C // 4, acc_body, 0)
        pltpu.async_copy(
            out_v, out_hbm.at[:, pl.ds(base + ch * _C, _C)], osem_b[buf]
        )

    def wait_out(ch, buf):
        pltpu.make_async_copy(
            out_b[buf], out_hbm.at[:, pl.ds(base + ch * _C, _C)], osem_b[buf]
        ).wait()

    prep(0, 0)

    def pair_body(h, carry):
        g0 = 2 * h
        prep(g0 + 1, 1)
        wait_gather(0)

        @pl.when(h >= 1)
        def _():
            wait_out(g0 - 2, 0)

        acc(g0, 0)

        @pl.when(h < _NCH // 2 - 1)
        def _():
            prep(g0 + 2, 0)

        wait_gather(1)

        @pl.when(h >= 1)
        def _():
            wait_out(g0 - 1, 1)

        acc(g0 + 1, 1)
        return carry

    lax.fori_loop(0, _NCH // 2, pair_body, 0)
    wait_out(_NCH - 2, 0)
    wait_out(_NCH - 1, 1)


_sc_hash = functools.partial(
    pl.kernel,
    out_type=jax.ShapeDtypeStruct((24, _N), jnp.float32),
    mesh=plsc.VectorSubcoreMesh(
        core_axis_name="c", subcore_axis_name="s", num_cores=_NC, num_subcores=_NS
    ),
    compiler_params=pltpu.CompilerParams(
        needs_layout_passes=False, use_tc_tiling_on_sc=False
    ),
    scratch_types=[
        pltpu.VMEM((_RPW,), jnp.float32),
        pltpu.VMEM((_RPW,), jnp.float32),
        pltpu.VMEM((_RPW,), jnp.float32),
        pltpu.VMEM((_NG * _C,), jnp.int32),
        pltpu.VMEM((_NG * _C,), jnp.int32),
        pltpu.VMEM((_S * _C,), jnp.float32),
        pltpu.VMEM((_S * _C,), jnp.float32),
        pltpu.VMEM((_NG * _C, 8), jnp.float32),
        pltpu.VMEM((_NG * _C, 8), jnp.float32),
        pltpu.VMEM((24, _C), jnp.float32),
        pltpu.VMEM((24, _C), jnp.float32),
        pltpu.SemaphoreType.DMA,
        pltpu.SemaphoreType.DMA,
        pltpu.SemaphoreType.DMA,
        pltpu.SemaphoreType.DMA,
    ],
)(_sc_body)


def kernel(view_dirs, ray_origins, table):
    coords = _coords_tc(view_dirs.T, ray_origins.T)
    xs, ys, zs = coords[0], coords[1], coords[2]
    tp = table.T.reshape(4, _NT, 128).transpose(1, 0, 2).reshape(_NT * 512)
    table2 = _ilv_sc(tp).reshape(_NT * 128, 8)
    out = _sc_hash(xs, ys, zs, table2)
    return out.T


# speedup vs baseline: 1.3971x; 1.0754x over previous
import functools

import jax
import jax.numpy as jnp
import numpy as np
from jax import lax
from jax.experimental import pallas as pl
from jax.experimental.pallas import tpu as pltpu
from jax.experimental.pallas import tpu_sc as plsc

_N = 262144
_NC, _NS, _L = 2, 16, 16
_NW = _NC * _NS
_RPW = _N // _NW
_C = 64
_NCH = _RPW // _C
_S = 48
_RADIUS = 500.0

_LEVELS = (
    (16, 0, 4096, False),
    (32, 4096, 32768, False),
    (64, 36864, 262144, False),
    (128, 299008, 524288, True),
    (256, 823296, 524288, True),
    (512, 1347584, 524288, True),
)
_P1 = np.int32(2654435761 - (1 << 32))
_P2 = np.int32(805459861)

_GSLOT = {}
_PSLOT = {}
_NG = 0
for _l, (_R, _off, _size, _hashed) in enumerate(_LEVELS):
    if _hashed:
        for _corner in range(8):
            _GSLOT[(_l, _corner)] = (_NG + _corner, 0)
        _NG += 8
    else:
        for _p in range(4):
            _PSLOT[(_l, _p)] = _NG + _p
            _GSLOT[(_l, 2 * _p)] = (_NG + _p, 0)
            _GSLOT[(_l, 2 * _p + 1)] = (_NG + _p, 4)
        _NG += 4


def _coords_body(vd_ref, ro_ref, out_ref):
    vd = vd_ref[...]
    ro = ro_ref[...]
    b = 2.0 * jnp.sum(vd * ro, axis=0, keepdims=True)
    c = jnp.sum(ro * ro, axis=0, keepdims=True) - _RADIUS * _RADIUS
    disc = b * b - 4.0 * c
    sq = jnp.sqrt(jnp.maximum(disc, 0.0))
    t = 0.5 * (sq - b)
    pts = ro + t * vd
    out_ref[...] = jnp.clip((pts + _RADIUS) / (2.0 * _RADIUS), 0.0, 1.0)


_BT = 8192


def _coords_tc(vd_t, ro_t):
    return pl.pallas_call(
        _coords_body,
        grid=(_N // _BT,),
        in_specs=[
            pl.BlockSpec((3, _BT), lambda i: (0, i)),
            pl.BlockSpec((3, _BT), lambda i: (0, i)),
        ],
        out_specs=pl.BlockSpec((3, _BT), lambda i: (0, i)),
        out_shape=jax.ShapeDtypeStruct((3, _N), jnp.float32),
    )(vd_t, ro_t)


_NT = 14624
_TPW = _NT // _NW
_TCH = 16
_NFULL = _TPW // _TCH
_TTAIL = _TPW - _NFULL * _TCH


def _ilv_body(tp_hbm, out_hbm, in0, in1, ov0, ov1, isem0, isem1, osem0, osem1):
    wid = lax.axis_index("s") * _NC + lax.axis_index("c")
    k0 = wid * _TPW
    lane = lax.iota(jnp.int32, _L)
    in_b = (in0, in1)
    ov_b = (ov0, ov1)
    isem_b = (isem0, isem1)
    osem_b = (osem0, osem1)

    def src(ch, nt):
        return tp_hbm.at[pl.ds((k0 + ch * _TCH) * 512, nt * 512)]

    def srcx(ch, nt):
        knext = jnp.minimum(k0 + ch * _TCH + nt, _NT - 1)
        return tp_hbm.at[pl.ds(knext * 512, 512)]

    def dst(ch, nt, buf):
        return out_hbm.at[pl.ds((k0 + ch * _TCH) * 1024, nt * 1024)]

    def ilv(buf, nt):
        in_v = in_b[buf]
        ov = ov_b[buf]

        def tile_body(t, c2):
            t512 = t * 512
            o512 = t * 1024
            for c in range(4):
                ibase = t512 + c * 128
                obase = o512 + 8 * lane + c
                nxt = in_v[pl.ds(t512 + 512 + c * 128, _L)]
                nxt0 = jnp.full((_L,), nxt[0], jnp.float32)
                for j in range(8):
                    v = in_v[pl.ds(ibase + 16 * j, _L)]
                    plsc.store_scatter(ov, [obase + 128 * j], v)
                    v2 = in_v[pl.ds(ibase + 16 * j + 1, _L)]
                    if j == 7:
                        v2 = jnp.where(lane == 15, nxt0, v2)
                    plsc.store_scatter(ov, [obase + 128 * j + 4], v2)
            return c2

        lax.fori_loop(0, nt, tile_body, 0)

    def prep(ch, buf, nt):
        pltpu.async_copy(src(ch, nt), in_b[buf].at[pl.ds(0, nt * 512)],
                         isem_b[buf])
        pltpu.async_copy(srcx(ch, nt), in_b[buf].at[pl.ds(nt * 512, 512)],
                         isem_b[buf])

    def wait_in(ch, buf, nt):
        pltpu.make_async_copy(src(ch, nt), in_b[buf].at[pl.ds(0, nt * 512)],
                              isem_b[buf]).wait()
        pltpu.make_async_copy(srcx(ch, nt), in_b[buf].at[pl.ds(nt * 512, 512)],
                              isem_b[buf]).wait()

    def put(ch, buf, nt):
        pltpu.async_copy(ov_b[buf].at[pl.ds(0, nt * 1024)], dst(ch, nt, buf),
                         osem_b[buf])

    def wait_put(ch, buf, nt):
        pltpu.make_async_copy(ov_b[buf].at[pl.ds(0, nt * 1024)],
                              dst(ch, nt, buf), osem_b[buf]).wait()

    prep(0, 0, _TCH)

    def pair_body(h, carry):
        g0 = 2 * h
        prep(g0 + 1, 1, _TCH)
        wait_in(g0, 0, _TCH)

        @pl.when(h >= 1)
        def _():
            wait_put(g0 - 2, 0, _TCH)

        ilv(0, _TCH)
        put(g0, 0, _TCH)

        @pl.when(h < _NFULL // 2 - 1)
        def _():
            prep(g0 + 2, 0, _TCH)

        wait_in(g0 + 1, 1, _TCH)

        @pl.when(h >= 1)
        def _():
            wait_put(g0 - 1, 1, _TCH)

        ilv(1, _TCH)
        put(g0 + 1, 1, _TCH)
        return carry

    lax.fori_loop(0, _NFULL // 2, pair_body, 0)
    wait_put(_NFULL - 2, 0, _TCH)
    wait_put(_NFULL - 1, 1, _TCH)
    prep(_NFULL, 0, _TTAIL)
    wait_in(_NFULL, 0, _TTAIL)
    ilv(0, _TTAIL)
    put(_NFULL, 0, _TTAIL)
    wait_put(_NFULL, 0, _TTAIL)


_ilv_sc = functools.partial(
    pl.kernel,
    out_type=jax.ShapeDtypeStruct((_NT * 1024,), jnp.float32),
    mesh=plsc.VectorSubcoreMesh(
        core_axis_name="c", subcore_axis_name="s", num_cores=_NC, num_subcores=_NS
    ),
    compiler_params=pltpu.CompilerParams(
        needs_layout_passes=False, use_tc_tiling_on_sc=False
    ),
    scratch_types=[
        pltpu.VMEM(((_TCH + 1) * 512,), jnp.float32),
        pltpu.VMEM(((_TCH + 1) * 512,), jnp.float32),
        pltpu.VMEM((_TCH * 1024,), jnp.float32),
        pltpu.VMEM((_TCH * 1024,), jnp.float32),
        pltpu.SemaphoreType.DMA,
        pltpu.SemaphoreType.DMA,
        pltpu.SemaphoreType.DMA,
        pltpu.SemaphoreType.DMA,
    ],
)(_ilv_body)


def _sc_body(xs_hbm, ys_hbm, zs_hbm, table_hbm, out_hbm,
             xs_v, ys_v, zs_v, l0_v, idx0, idx1, w0, w1, rows0, rows1,
             out0, out1, gsem0, gsem1, osem0, osem1):
    wid = lax.axis_index("s") * _NC + lax.axis_index("c")
    lane = lax.iota(jnp.int32, _L)
    rep4 = lane >> 2
    col = lane & 3

    base = wid * _RPW
    pltpu.sync_copy(xs_hbm.at[pl.ds(base, _RPW)], xs_v)
    pltpu.sync_copy(ys_hbm.at[pl.ds(base, _RPW)], ys_v)
    pltpu.sync_copy(zs_hbm.at[pl.ds(base, _RPW)], zs_v)
    pltpu.sync_copy(table_hbm.at[pl.ds(0, 4096), :], l0_v)

    idx_b = (idx0, idx1)
    w_b = (w0, w1)
    rows_b = (rows0, rows1)
    out_b = (out0, out1)
    gsem_b = (gsem0, gsem1)
    osem_b = (osem0, osem1)

    def prep(ch, buf):
        idx_v = idx_b[buf]
        w_v = w_b[buf]

        def iw_body(i, c2):
            o = i * _L
            og = ch * _C + o
            x = xs_v[pl.ds(og, _L)]
            y = ys_v[pl.ds(og, _L)]
            z = zs_v[pl.ds(og, _L)]
            for l, (R, off, size, hashed) in enumerate(_LEVELS):
                scale = jnp.float32(R - 1)
                px = x * scale
                py = y * scale
                pz = z * scale
                cx = jnp.minimum(px.astype(jnp.int32), R - 2)
                cy = jnp.minimum(py.astype(jnp.int32), R - 2)
                cz = jnp.minimum(pz.astype(jnp.int32), R - 2)
                fx = px - cx.astype(jnp.float32)
                fy = py - cy.astype(jnp.float32)
                fz = pz - cz.astype(jnp.float32)
                wx = (1.0 - fx, fx)
                wy = (1.0 - fy, fy)
                wz = (1.0 - fz, fz)
                if hashed:
                    hy0 = cy * _P1
                    hz0 = cz * _P2
                    hx = (cx, cx + 1)
                    hy = (hy0, hy0 + _P1)
                    hz = (hz0, hz0 + _P2)
                    mask = size - 1
                    for corner in range(8):
                        idx = (
                            (hx[corner & 1]
                             ^ hy[(corner >> 1) & 1]
                             ^ hz[(corner >> 2) & 1]) & mask
                        ) + off
                        gslot = _GSLOT[(l, corner)][0]
                        idx_v[pl.ds(gslot * _C + o, _L)] = idx
                else:
                    by0 = cy * R
                    bz0 = cz * (R * R)
                    by = (by0, by0 + R)
                    bz = (bz0, bz0 + R * R)
                    for p in range(4):
                        jp = cx + by[p & 1] + bz[(p >> 1) & 1] + off
                        idx_v[pl.ds(_PSLOT[(l, p)] * _C + o, _L)] = jp
                for corner in range(8):
                    dx, dy, dz = corner & 1, (corner >> 1) & 1, (corner >> 2) & 1
                    slot = l * 8 + corner
                    w_v[pl.ds(slot * _C + o, _L)] = wx[dx] * wy[dy] * wz[dz]
            return c2

        lax.fori_loop(0, _C // _L, iw_body, 0)

        pltpu.async_copy(
            table_hbm.at[idx_v.at[pl.ds(4 * _C, (_NG - 4) * _C)]],
            rows_b[buf],
            gsem_b[buf],
        )

    def wait_gather(buf):
        pltpu.make_async_copy(
            table_hbm.at[idx_b[buf].at[pl.ds(4 * _C, (_NG - 4) * _C)]],
            rows_b[buf],
            gsem_b[buf],
        ).wait()

    def acc(ch, buf):
        rows_v = rows_b[buf]
        idx_v = idx_b[buf]
        w_v = w_b[buf]
        out_v = out_b[buf]

        def acc_body(j, c2):
            r4 = 4 * j + rep4
            for l in range(6):
                a = jnp.zeros((_L,), jnp.float32)
                if l == 0:
                    for p in range(4):
                        ji = plsc.load_gather(idx_v, [p * _C + r4])
                        for dx in range(2):
                            s = l * 8 + 2 * p + dx
                            wv = plsc.load_gather(w_v, [s * _C + r4])
                            rv = plsc.load_gather(l0_v, [ji, col + 4 * dx])
                            a = a + wv * rv
                else:
                    for corner in range(8):
                        s = l * 8 + corner
                        gslot, coff = _GSLOT[(l, corner)]
                        wv = plsc.load_gather(w_v, [s * _C + r4])
                        rv = plsc.load_gather(
                            rows_v, [(gslot - 4) * _C + r4, col + coff]
                        )
                        a = a + wv * rv
                plsc.store_scatter(out_v, [4 * l + col, r4], a)
            return c2

        lax.fori_loop(0, _C // 4, acc_body, 0)
        pltpu.async_copy(
            out_v, out_hbm.at[:, pl.ds(base + ch * _C, _C)], osem_b[buf]
        )

    def wait_out(ch, buf):
        pltpu.make_async_copy(
            out_b[buf], out_hbm.at[:, pl.ds(base + ch * _C, _C)], osem_b[buf]
        ).wait()

    prep(0, 0)

    def pair_body(h, carry):
        g0 = 2 * h
        prep(g0 + 1, 1)
        wait_gather(0)

        @pl.when(h >= 1)
        def _():
            wait_out(g0 - 2, 0)

        acc(g0, 0)

        @pl.when(h < _NCH // 2 - 1)
        def _():
            prep(g0 + 2, 0)

        wait_gather(1)

        @pl.when(h >= 1)
        def _():
            wait_out(g0 - 1, 1)

        acc(g0 + 1, 1)
        return carry

    lax.fori_loop(0, _NCH // 2, pair_body, 0)
    wait_out(_NCH - 2, 0)
    wait_out(_NCH - 1, 1)


_sc_hash = functools.partial(
    pl.kernel,
    out_type=jax.ShapeDtypeStruct((24, _N), jnp.float32),
    mesh=plsc.VectorSubcoreMesh(
        core_axis_name="c", subcore_axis_name="s", num_cores=_NC, num_subcores=_NS
    ),
    compiler_params=pltpu.CompilerParams(
        needs_layout_passes=False, use_tc_tiling_on_sc=False
    ),
    scratch_types=[
        pltpu.VMEM((_RPW,), jnp.float32),
        pltpu.VMEM((_RPW,), jnp.float32),
        pltpu.VMEM((_RPW,), jnp.float32),
        pltpu.VMEM((4096, 8), jnp.float32),
        pltpu.VMEM((_NG * _C,), jnp.int32),
        pltpu.VMEM((_NG * _C,), jnp.int32),
        pltpu.VMEM((_S * _C,), jnp.float32),
        pltpu.VMEM((_S * _C,), jnp.float32),
        pltpu.VMEM(((_NG - 4) * _C, 8), jnp.float32),
        pltpu.VMEM(((_NG - 4) * _C, 8), jnp.float32),
        pltpu.VMEM((24, _C), jnp.float32),
        pltpu.VMEM((24, _C), jnp.float32),
        pltpu.SemaphoreType.DMA,
        pltpu.SemaphoreType.DMA,
        pltpu.SemaphoreType.DMA,
        pltpu.SemaphoreType.DMA,
    ],
)(_sc_body)


def kernel(view_dirs, ray_origins, table):
    coords = _coords_tc(view_dirs.T, ray_origins.T)
    xs, ys, zs = coords[0], coords[1], coords[2]
    tp = table.T.reshape(4, _NT, 128).transpose(1, 0, 2).reshape(_NT * 512)
    table2 = _ilv_sc(tp).reshape(_NT * 128, 8)
    out = _sc_hash(xs, ys, zs, table2)
    return out.T
